# Initial kernel scaffold; baseline (speedup 1.0000x reference)
import functools

import jax
import jax.numpy as jnp
from jax import lax
from jax.experimental import pallas as pl
from jax.experimental.pallas import tpu as pltpu
from jax.experimental.pallas import tpu_sc as plsc

F = 26
V = 100000
D = 32
B = 16384

NW = 32
ROWS = B * F
RPW = ROWS // NW
C = 832
NCH = RPW // C
L = 16


@functools.partial(
    pl.kernel,
    out_type=jax.ShapeDtypeStruct((ROWS, D), jnp.float32),
    mesh=plsc.VectorSubcoreMesh(core_axis_name="c", subcore_axis_name="s"),
    scratch_types=[
        pltpu.VMEM((C,), jnp.int32),
        pltpu.VMEM((C,), jnp.int32),
        pltpu.VMEM((C, D), jnp.float32),
        pltpu.SemaphoreType.DMA,
    ],
)
def _embed_gather(x_hbm, tab_hbm, out_hbm, off_v, idx_v, rows_v, gsem):
    wid = lax.axis_index("s") * 2 + lax.axis_index("c")
    base = wid * RPW

    def off_body(j, _):
        pos = j * L + lax.broadcasted_iota(jnp.int32, (L,), 0)
        off_v[pl.ds(j * L, L)] = lax.rem(pos, F) * V
        return 0

    lax.fori_loop(0, C // L, off_body, 0)

    def chunk_body(c, _):
        s = base + c * C
        pltpu.sync_copy(x_hbm.at[pl.ds(s, C)], idx_v)

        def add_body(j, _):
            sl = pl.ds(j * L, L)
            idx_v[sl] = idx_v[sl] + off_v[sl]
            return 0

        lax.fori_loop(0, C // L, add_body, 0)
        pltpu.async_copy(tab_hbm.at[idx_v], rows_v, gsem).wait()
        pltpu.sync_copy(rows_v, out_hbm.at[pl.ds(s, C)])
        return 0

    lax.fori_loop(0, NCH, chunk_body, 0)


def kernel(x, tables):
    x_flat = x.reshape(ROWS)
    tab = tables.reshape(F * V, D)
    out = _embed_gather(x_flat, tab)
    return out.reshape(B, F * D)


# speedup vs baseline: 1.1958x; 1.1958x over previous
import functools

import jax
import jax.numpy as jnp
from jax import lax
from jax.experimental import pallas as pl
from jax.experimental.pallas import tpu as pltpu
from jax.experimental.pallas import tpu_sc as plsc

F = 26
V = 100000
D = 32
B = 16384

NW = 32
ROWS = B * F
RPW = ROWS // NW
C = 832
NCH = RPW // C
L = 16


@functools.partial(
    pl.kernel,
    out_type=jax.ShapeDtypeStruct((ROWS, D), jnp.float32),
    mesh=plsc.VectorSubcoreMesh(core_axis_name="c", subcore_axis_name="s"),
    scratch_types=[
        pltpu.VMEM((C,), jnp.int32),
        pltpu.VMEM((C,), jnp.int32),
        pltpu.VMEM((C, D), jnp.float32),
        pltpu.SemaphoreType.DMA,
    ],
    compiler_params=pltpu.CompilerParams(use_tc_tiling_on_sc=False),
)
def _embed_gather(x_hbm, tab_hbm, out_hbm, off_v, idx_v, rows_v, gsem):
    wid = lax.axis_index("s") * 2 + lax.axis_index("c")
    base = wid * RPW

    def off_body(j, _):
        pos = j * L + lax.broadcasted_iota(jnp.int32, (L,), 0)
        off_v[pl.ds(j * L, L)] = lax.rem(pos, F) * V
        return 0

    lax.fori_loop(0, C // L, off_body, 0)

    def chunk_body(c, _):
        s = base + c * C
        pltpu.sync_copy(x_hbm.at[pl.ds(s, C)], idx_v)

        def add_body(j, _):
            sl = pl.ds(j * L, L)
            idx_v[sl] = idx_v[sl] + off_v[sl]
            return 0

        lax.fori_loop(0, C // L, add_body, 0)
        pltpu.async_copy(tab_hbm.at[idx_v], rows_v, gsem).wait()
        pltpu.sync_copy(rows_v, out_hbm.at[pl.ds(s, C)])
        return 0

    lax.fori_loop(0, NCH, chunk_body, 0)


def kernel(x, tables):
    x_flat = x.reshape(ROWS)
    tab = tables.reshape(F * V, D)
    out = _embed_gather(x_flat, tab)
    return out.reshape(B, F * D)


# speedup vs baseline: 1.2138x; 1.0151x over previous
import functools

import jax
import jax.numpy as jnp
from jax import lax
from jax.experimental import pallas as pl
from jax.experimental.pallas import tpu as pltpu
from jax.experimental.pallas import tpu_sc as plsc

F = 26
V = 100000
D = 32
B = 16384

NW = 32
ROWS = B * F
RPW = ROWS // NW
C = 512
NCH = RPW // C
NBUF = 6
DEPTH = 5
L = 16


@functools.partial(
    pl.kernel,
    out_type=jax.ShapeDtypeStruct((ROWS, D), jnp.float32),
    mesh=plsc.VectorSubcoreMesh(core_axis_name="c", subcore_axis_name="s"),
    scratch_types=(
        [pltpu.VMEM((RPW,), jnp.int32),
         pltpu.VMEM((NBUF, C, D), jnp.float32)]
        + [pltpu.SemaphoreType.DMA] * (2 * NBUF)
    ),
    compiler_params=pltpu.CompilerParams(use_tc_tiling_on_sc=False),
)
def _embed_gather(x_hbm, tab_hbm, out_hbm, idx_all, rows_v, *sems):
    gsems = sems[:NBUF]
    osems = sems[NBUF:]
    wid = lax.axis_index("s") * 2 + lax.axis_index("c")
    base = wid * RPW

    pltpu.sync_copy(x_hbm.at[pl.ds(base, RPW)], idx_all)

    def add_body(j, _):
        sl = pl.ds(j * L, L)
        pos = j * L + lax.broadcasted_iota(jnp.int32, (L,), 0)
        idx_all[sl] = idx_all[sl] + lax.rem(pos, F) * V
        return 0

    lax.fori_loop(0, RPW // L, add_body, 0)

    def start_gather(c):
        b = c % NBUF
        return pltpu.async_copy(
            tab_hbm.at[idx_all.at[pl.ds(c * C, C)]], rows_v.at[b], gsems[b])

    def start_store(c):
        b = c % NBUF
        return pltpu.async_copy(
            rows_v.at[b], out_hbm.at[pl.ds(base + c * C, C)], osems[b])

    gh = [None] * NCH
    sh = [None] * NCH
    for c in range(NCH):
        if c >= NBUF:
            sh[c - NBUF].wait()
        gh[c] = start_gather(c)
        d = c - DEPTH
        if d >= 0:
            gh[d].wait()
            sh[d] = start_store(d)
    for d in range(NCH - DEPTH, NCH):
        gh[d].wait()
        sh[d] = start_store(d)
    for d in range(NCH - NBUF, NCH):
        sh[d].wait()


def kernel(x, tables):
    x_flat = x.reshape(ROWS)
    tab = tables.reshape(F * V, D)
    out = _embed_gather(x_flat, tab)
    return out.reshape(B, F * D)


# speedup vs baseline: 4.1810x; 3.4444x over previous
import functools

import jax
import jax.numpy as jnp
from jax import lax
from jax.experimental import pallas as pl
from jax.experimental.pallas import tpu as pltpu
from jax.experimental.pallas import tpu_sc as plsc

F = 26
V = 100000
D = 32
B = 16384

NW = 32
TT = F * D
RPT = TT // NW
HB = B // 2
L = 16


@functools.partial(
    pl.kernel,
    out_type=jax.ShapeDtypeStruct((TT, B), jnp.float32),
    mesh=plsc.VectorSubcoreMesh(core_axis_name="c", subcore_axis_name="s"),
    scratch_types=[
        pltpu.VMEM((V,), jnp.float32),
        pltpu.VMEM((HB,), jnp.int32),
        pltpu.VMEM((HB,), jnp.float32),
    ],
    compiler_params=pltpu.CompilerParams(needs_layout_passes=False),
)
def _embed_rows(xt_hbm, tabt_hbm, out_hbm, row_v, idx_v, val_v):
    w = lax.axis_index("s") * 2 + lax.axis_index("c")

    def row_body(k, _):
        r = w * RPT + k
        f = r // D
        d = r - f * D
        pltpu.sync_copy(tabt_hbm.at[f, d], row_v)

        for h in range(2):
            pltpu.sync_copy(xt_hbm.at[f, pl.ds(h * HB, HB)], idx_v)

            def g16(j, _):
                sl = pl.ds(j * L, L)
                val_v[sl] = plsc.load_gather(row_v, [idx_v[sl]])
                return 0

            lax.fori_loop(0, HB // L, g16, 0)
            pltpu.sync_copy(val_v, out_hbm.at[r, pl.ds(h * HB, HB)])
        return 0

    lax.fori_loop(0, RPT, row_body, 0)


def kernel(x, tables):
    xt = x.T
    tabt = jnp.transpose(tables, (0, 2, 1))
    out = _embed_rows(xt, tabt)
    return out.T


# speedup vs baseline: 4.2513x; 1.0168x over previous
import functools

import jax
import jax.numpy as jnp
from jax import lax
from jax.experimental import pallas as pl
from jax.experimental.pallas import tpu as pltpu
from jax.experimental.pallas import tpu_sc as plsc

F = 26
V = 100000
D = 32
B = 16384

NW = 32
TT = F * D
RPT = TT // NW
NCK = 4
CB = B // NCK
L = 16
UNROLL = 4


@functools.partial(
    pl.kernel,
    out_type=jax.ShapeDtypeStruct((TT, B), jnp.float32),
    mesh=plsc.VectorSubcoreMesh(core_axis_name="c", subcore_axis_name="s"),
    scratch_types=(
        [pltpu.VMEM((V,), jnp.float32),
         pltpu.VMEM((2, CB), jnp.int32),
         pltpu.VMEM((2, CB), jnp.float32)]
        + [pltpu.SemaphoreType.DMA] * 5
    ),
    compiler_params=pltpu.CompilerParams(needs_layout_passes=False),
)
def _embed_rows(xt_hbm, tabt_hbm, out_hbm, row_v, idx_v, val_v,
                rsem, xsem0, xsem1, vsem0, vsem1):
    xsems = (xsem0, xsem1)
    vsems = (vsem0, vsem1)
    w = lax.axis_index("s") * 2 + lax.axis_index("c")

    def row_body(k, prev_stores):
        r = w * RPT + k
        f = r // D
        d = r - f * D
        h_row = pltpu.async_copy(tabt_hbm.at[f, d], row_v, rsem)
        h_x = [None] * NCK
        h_x[0] = pltpu.async_copy(
            xt_hbm.at[f, pl.ds(0, CB)], idx_v.at[0], xsems[0])
        h_v = [None] * NCK
        for s in prev_stores:
            s.wait()
        h_row.wait()
        for c in range(NCK):
            if c + 1 < NCK:
                h_x[c + 1] = pltpu.async_copy(
                    xt_hbm.at[f, pl.ds((c + 1) * CB, CB)],
                    idx_v.at[(c + 1) % 2], xsems[(c + 1) % 2])
            h_x[c].wait()
            if c >= 2:
                h_v[c - 2].wait()
            p = c % 2

            def g64(j, _, p=p):
                base = j * (L * UNROLL)
                for u in range(UNROLL):
                    sl = pl.ds(base + u * L, L)
                    val_v[p, sl] = plsc.load_gather(row_v, [idx_v[p, sl]])
                return 0

            lax.fori_loop(0, CB // (L * UNROLL), g64, 0)
            h_v[c] = pltpu.async_copy(
                val_v.at[p], out_hbm.at[r, pl.ds(c * CB, CB)], vsems[p])
        return [h_v[NCK - 2], h_v[NCK - 1]]

    stores = []
    for k in range(RPT):
        stores = row_body(k, stores)
    for s in stores:
        s.wait()


def kernel(x, tables):
    xt = x.T
    tabt = jnp.transpose(tables, (0, 2, 1))
    out = _embed_rows(xt, tabt)
    return out.T


# speedup vs baseline: 4.2535x; 1.0005x over previous
import functools

import jax
import jax.numpy as jnp
from jax import lax
from jax.experimental import pallas as pl
from jax.experimental.pallas import tpu as pltpu
from jax.experimental.pallas import tpu_sc as plsc

F = 26
V = 100000
D = 32
B = 16384

NW = 32
TT = F * D
RPT = TT // NW
NCK = 4
CB = B // NCK
L = 16
UNROLL = 4


@functools.partial(
    pl.kernel,
    out_type=jax.ShapeDtypeStruct((TT, B), jnp.float32),
    mesh=plsc.VectorSubcoreMesh(core_axis_name="c", subcore_axis_name="s"),
    scratch_types=(
        [pltpu.VMEM((V,), jnp.float32),
         pltpu.VMEM((2, CB), jnp.int32),
         pltpu.VMEM((2, CB), jnp.float32)]
        + [pltpu.SemaphoreType.DMA] * 5
    ),
    compiler_params=pltpu.CompilerParams(needs_layout_passes=False),
)
def _embed_rows(xt_hbm, tabt_hbm, out_hbm, row_v, idx_v, val_v,
                rsem, xsem0, xsem1, vsem0, vsem1):
    xsems = (xsem0, xsem1)
    vsems = (vsem0, vsem1)
    w = lax.axis_index("s") * 2 + lax.axis_index("c")
    grp = w // 8
    j = w - grp * 8

    def row_body(k, prev_stores):
        o = grp * RPT + k
        f = o // 4
        g = o - f * 4
        d = g * 8 + j
        r = f * D + d
        h_row = pltpu.async_copy(tabt_hbm.at[f, d], row_v, rsem)
        h_x = [None] * NCK
        h_x[0] = pltpu.async_copy(
            xt_hbm.at[f, pl.ds(0, CB)], idx_v.at[0], xsems[0])
        h_v = [None] * NCK
        for s in prev_stores:
            s.wait()
        h_row.wait()
        for c in range(NCK):
            if c + 1 < NCK:
                h_x[c + 1] = pltpu.async_copy(
                    xt_hbm.at[f, pl.ds((c + 1) * CB, CB)],
                    idx_v.at[(c + 1) % 2], xsems[(c + 1) % 2])
            h_x[c].wait()
            if c >= 2:
                h_v[c - 2].wait()
            p = c % 2

            def g64(j, _, p=p):
                base = j * (L * UNROLL)
                for u in range(UNROLL):
                    sl = pl.ds(base + u * L, L)
                    val_v[p, sl] = plsc.load_gather(row_v, [idx_v[p, sl]])
                return 0

            lax.fori_loop(0, CB // (L * UNROLL), g64, 0)
            h_v[c] = pltpu.async_copy(
                val_v.at[p], out_hbm.at[r, pl.ds(c * CB, CB)], vsems[p])
        return [h_v[NCK - 2], h_v[NCK - 1]]

    stores = []
    for k in range(RPT):
        stores = row_body(k, stores)
    for s in stores:
        s.wait()


def kernel(x, tables):
    xt = x.T
    tabt = jnp.transpose(tables, (0, 2, 1))
    out = _embed_rows(xt, tabt)
    return out.T
